# Initial kernel scaffold; baseline (speedup 1.0000x reference)
import jax
import jax.numpy as jnp
from jax.experimental import pallas as pl


def kernel(x, edge_index, edge_weight, prev_hidden_state, W_conv_z, b_conv_z, W_conv_r, b_conv_r, W_conv_h, b_conv_h, W_lin_z, b_lin_z, W_lin_r, b_lin_r, W_lin_h, b_lin_h, W_out, b_out):
    raise NotImplementedError("write your pallas kernel here")


# speedup vs baseline: 16.4848x; 16.4848x over previous
import functools

import jax
import jax.numpy as jnp
from jax import lax
from jax.experimental import pallas as pl
from jax.experimental.pallas import tpu as pltpu
from jax.experimental.pallas import tpu_sc as plsc

N = 10000
E = 320000
F_IN = 128
F_OUT = 32
F3 = 3 * F_OUT
NP = 10240
NC = 2
NS = 16
NW = NC * NS

EPW_A = E // NW


def _deg_body(col_hbm, w_hbm, degp_hbm, col_v, w_v, acc_v):
    wid = lax.axis_index("s") * NC + lax.axis_index("c")
    base = wid * EPW_A
    pltpu.sync_copy(col_hbm.at[pl.ds(base, EPW_A)], col_v)
    pltpu.sync_copy(w_hbm.at[pl.ds(base, EPW_A)], w_v)

    zero16 = jnp.zeros((16,), jnp.float32)

    @pl.loop(0, NP // 16)
    def _zero(j):
        acc_v[pl.ds(j * 16, 16)] = zero16

    @pl.loop(0, EPW_A // 16)
    def _edges(g):
        b = g * 16
        c16 = col_v[pl.ds(b, 16)]
        w16 = w_v[pl.ds(b, 16)]
        plsc.addupdate_scatter(acc_v, [c16], w16)

    pltpu.sync_copy(acc_v, degp_hbm.at[wid])


def _deg_partials(col, w):
    mesh = plsc.VectorSubcoreMesh(core_axis_name="c", subcore_axis_name="s")
    return pl.kernel(
        _deg_body,
        compiler_params=pltpu.CompilerParams(needs_layout_passes=False, use_tc_tiling_on_sc=False),
        out_type=jax.ShapeDtypeStruct((NW, NP), jnp.float32),
        mesh=mesh,
        scratch_types=[
            pltpu.VMEM((EPW_A,), jnp.int32),
            pltpu.VMEM((EPW_A,), jnp.float32),
            pltpu.VMEM((NP,), jnp.float32),
        ],
    )(col, w)


BN_B = 512


def _xwn_body(xT_ref, degp_ref, wcatT_ref, xwnT_ref, dis_ref):
    deg = jnp.sum(degp_ref[...], axis=0, keepdims=True) + 1.0
    dis = lax.rsqrt(deg)
    xwT = jnp.dot(wcatT_ref[...], xT_ref[...], preferred_element_type=jnp.float32)
    xwnT_ref[...] = xwT * dis
    dis_ref[...] = dis


def _xwn(xT, degp, wcatT):
    grid = (NP // BN_B,)
    return pl.pallas_call(
        _xwn_body,
        grid=grid,
        in_specs=[
            pl.BlockSpec((F_IN, BN_B), lambda i: (0, i)),
            pl.BlockSpec((NW, BN_B), lambda i: (0, i)),
            pl.BlockSpec((F3, F_IN), lambda i: (0, 0)),
        ],
        out_specs=[
            pl.BlockSpec((F3, BN_B), lambda i: (0, i)),
            pl.BlockSpec((1, BN_B), lambda i: (0, i)),
        ],
        out_shape=[
            jax.ShapeDtypeStruct((F3, NP), jnp.float32),
            jax.ShapeDtypeStruct((1, NP), jnp.float32),
        ],
    )(xT, degp, wcatT)


SPLIT_F = 16
SPLIT_E = NW // SPLIT_F
FPW = F3 // SPLIT_F
EPW_C = E // SPLIT_E
CH_C = 2000
NCH_C = EPW_C // CH_C


def _msg_body(row_hbm, col_hbm, w_hbm, xwnT_hbm, accp_hbm,
              table_v, acc_v, row_v, col_v, w_v):
    wid = lax.axis_index("s") * NC + lax.axis_index("c")
    fgroup = wid % SPLIT_F
    eslice = wid // SPLIT_F
    fbase = fgroup * FPW

    pltpu.sync_copy(xwnT_hbm.at[pl.ds(fbase, FPW)], table_v)

    zero16 = jnp.zeros((16,), jnp.float32)
    for f in range(FPW):
        @pl.loop(0, NP // 16)
        def _zero(j, f=f):
            acc_v[f, pl.ds(j * 16, 16)] = zero16

    ebase = eslice * EPW_C

    @pl.loop(0, NCH_C)
    def _chunk(ci):
        off = ebase + ci * CH_C
        pltpu.sync_copy(row_hbm.at[pl.ds(off, CH_C)], row_v)
        pltpu.sync_copy(col_hbm.at[pl.ds(off, CH_C)], col_v)
        pltpu.sync_copy(w_hbm.at[pl.ds(off, CH_C)], w_v)

        @pl.loop(0, CH_C // 16)
        def _group(g):
            b = g * 16
            r16 = row_v[pl.ds(b, 16)]
            c16 = col_v[pl.ds(b, 16)]
            w16 = w_v[pl.ds(b, 16)]
            for f in range(FPW):
                t = plsc.load_gather(table_v.at[f], [r16])
                plsc.addupdate_scatter(acc_v.at[f], [c16], t * w16)

    pltpu.sync_copy(acc_v, accp_hbm.at[eslice, pl.ds(fbase, FPW)])


def _msg_partials(row, col, w, xwnT):
    mesh = plsc.VectorSubcoreMesh(core_axis_name="c", subcore_axis_name="s")
    return pl.kernel(
        _msg_body,
        compiler_params=pltpu.CompilerParams(needs_layout_passes=False, use_tc_tiling_on_sc=False),
        out_type=jax.ShapeDtypeStruct((SPLIT_E, F3, NP), jnp.float32),
        mesh=mesh,
        scratch_types=[
            pltpu.VMEM((FPW, NP), jnp.float32),
            pltpu.VMEM((FPW, NP), jnp.float32),
            pltpu.VMEM((CH_C,), jnp.int32),
            pltpu.VMEM((CH_C,), jnp.int32),
            pltpu.VMEM((CH_C,), jnp.float32),
        ],
    )(row, col, w, xwnT)


BN_D = 512


def _gates_body(accp_ref, xwnT_ref, dis_ref, hT_ref, bcat_ref,
                wz1_ref, wz2_ref, bz_ref, wr1_ref, wr2_ref, br_ref,
                wh1_ref, wh2_ref, bh_ref, wo_ref, bo_ref,
                hnT_ref, yT_ref):
    acc = accp_ref[0] + accp_ref[1]
    dis = dis_ref[...]
    S = dis * (acc + xwnT_ref[...]) + bcat_ref[...]
    cz = S[0:F_OUT]
    cr = S[F_OUT:2 * F_OUT]
    ch = S[2 * F_OUT:F3]
    H = hT_ref[...]

    def mm(a, b):
        return jnp.dot(a, b, preferred_element_type=jnp.float32)

    Z = jax.nn.sigmoid(mm(wz1_ref[...], cz) + mm(wz2_ref[...], H) + bz_ref[...])
    R = jax.nn.sigmoid(mm(wr1_ref[...], cr) + mm(wr2_ref[...], H) + br_ref[...])
    Ht = jnp.tanh(mm(wh1_ref[...], ch) + mm(wh2_ref[...], H * R) + bh_ref[...])
    Hn = Z * H + (1.0 - Z) * Ht
    hnT_ref[...] = Hn
    yT_ref[...] = mm(wo_ref[...], jnp.maximum(Hn, 0.0)) + bo_ref[...]


def _gates(accp, xwnT, dis, hT, bcat, wz1, wz2, bz, wr1, wr2, br,
           wh1, wh2, bh, wo, bo):
    grid = (NP // BN_D,)
    full = lambda shape: pl.BlockSpec(shape, lambda i: tuple(0 for _ in shape))
    return pl.pallas_call(
        _gates_body,
        grid=grid,
        in_specs=[
            pl.BlockSpec((SPLIT_E, F3, BN_D), lambda i: (0, 0, i)),
            pl.BlockSpec((F3, BN_D), lambda i: (0, i)),
            pl.BlockSpec((1, BN_D), lambda i: (0, i)),
            pl.BlockSpec((F_OUT, BN_D), lambda i: (0, i)),
            full((F3, 1)),
            full((F_OUT, F_OUT)), full((F_OUT, F_OUT)), full((F_OUT, 1)),
            full((F_OUT, F_OUT)), full((F_OUT, F_OUT)), full((F_OUT, 1)),
            full((F_OUT, F_OUT)), full((F_OUT, F_OUT)), full((F_OUT, 1)),
            full((1, F_OUT)), full((1, 1)),
        ],
        out_specs=[
            pl.BlockSpec((F_OUT, BN_D), lambda i: (0, i)),
            pl.BlockSpec((1, BN_D), lambda i: (0, i)),
        ],
        out_shape=[
            jax.ShapeDtypeStruct((F_OUT, NP), jnp.float32),
            jax.ShapeDtypeStruct((1, NP), jnp.float32),
        ],
    )(accp, xwnT, dis, hT, bcat, wz1, wz2, bz, wr1, wr2, br,
      wh1, wh2, bh, wo, bo)


def kernel(x, edge_index, edge_weight, prev_hidden_state,
           W_conv_z, b_conv_z, W_conv_r, b_conv_r, W_conv_h, b_conv_h,
           W_lin_z, b_lin_z, W_lin_r, b_lin_r, W_lin_h, b_lin_h,
           W_out, b_out):
    row = edge_index[0]
    col = edge_index[1]

    xT = jnp.zeros((F_IN, NP), jnp.float32).at[:, :N].set(x.T)
    hT = jnp.zeros((F_OUT, NP), jnp.float32).at[:, :N].set(prev_hidden_state.T)
    wcatT = jnp.concatenate([W_conv_z, W_conv_r, W_conv_h], axis=1).T
    bcat = jnp.concatenate([b_conv_z, b_conv_r, b_conv_h]).reshape(F3, 1)

    degp = _deg_partials(col, edge_weight)
    xwnT, dis = _xwn(xT, degp, wcatT)
    accp = _msg_partials(row, col, edge_weight, xwnT)

    hnT, yT = _gates(
        accp, xwnT, dis, hT, bcat,
        W_lin_z[:F_OUT].T, W_lin_z[F_OUT:].T, b_lin_z.reshape(F_OUT, 1),
        W_lin_r[:F_OUT].T, W_lin_r[F_OUT:].T, b_lin_r.reshape(F_OUT, 1),
        W_lin_h[:F_OUT].T, W_lin_h[F_OUT:].T, b_lin_h.reshape(F_OUT, 1),
        W_out.T, b_out.reshape(1, 1),
    )
    H_new = hnT[:, :N].T
    y = yT[:, :N].T
    return y, H_new


# speedup vs baseline: 43.1422x; 2.6171x over previous
import functools

import jax
import jax.numpy as jnp
from jax import lax
from jax.experimental import pallas as pl
from jax.experimental.pallas import tpu as pltpu
from jax.experimental.pallas import tpu_sc as plsc

N = 10000
E = 320000
F_IN = 128
F_OUT = 32
F3 = 3 * F_OUT
NP = 10240
NC = 2
NS = 16
NW = NC * NS

EPW_A = E // NW


def _deg_body(col_hbm, w_hbm, degp_hbm, col_v, w_v, acc_v):
    wid = lax.axis_index("s") * NC + lax.axis_index("c")
    base = wid * EPW_A
    pltpu.sync_copy(col_hbm.at[pl.ds(base, EPW_A)], col_v)
    pltpu.sync_copy(w_hbm.at[pl.ds(base, EPW_A)], w_v)

    zero16 = jnp.zeros((16,), jnp.float32)

    @pl.loop(0, NP // 16)
    def _zero(j):
        acc_v[pl.ds(j * 16, 16)] = zero16

    @pl.loop(0, EPW_A // 16)
    def _edges(g):
        b = g * 16
        c16 = col_v[pl.ds(b, 16)]
        w16 = w_v[pl.ds(b, 16)]
        plsc.addupdate_scatter(acc_v, [c16], w16)

    pltpu.sync_copy(acc_v, degp_hbm.at[wid])


def _deg_partials(col, w):
    mesh = plsc.VectorSubcoreMesh(core_axis_name="c", subcore_axis_name="s")
    return pl.kernel(
        _deg_body,
        compiler_params=pltpu.CompilerParams(needs_layout_passes=False, use_tc_tiling_on_sc=False),
        out_type=jax.ShapeDtypeStruct((NW, NP), jnp.float32),
        mesh=mesh,
        scratch_types=[
            pltpu.VMEM((EPW_A,), jnp.int32),
            pltpu.VMEM((EPW_A,), jnp.float32),
            pltpu.VMEM((NP,), jnp.float32),
        ],
    )(col, w)


BN_B = 512


def _xwn_body(xT_ref, degp_ref, wcatT_ref, xwnT_ref, dis_ref):
    deg = jnp.sum(degp_ref[...], axis=0, keepdims=True) + 1.0
    dis = lax.rsqrt(deg)
    xwT = jnp.dot(wcatT_ref[...], xT_ref[...], preferred_element_type=jnp.float32)
    xwnT_ref[...] = xwT * dis
    dis_ref[...] = dis


def _xwn(xT, degp, wcatT):
    grid = (NP // BN_B,)
    return pl.pallas_call(
        _xwn_body,
        grid=grid,
        in_specs=[
            pl.BlockSpec((F_IN, BN_B), lambda i: (0, i)),
            pl.BlockSpec((NW, BN_B), lambda i: (0, i)),
            pl.BlockSpec((F3, F_IN), lambda i: (0, 0)),
        ],
        out_specs=[
            pl.BlockSpec((F3, BN_B), lambda i: (0, i)),
            pl.BlockSpec((1, BN_B), lambda i: (0, i)),
        ],
        out_shape=[
            jax.ShapeDtypeStruct((F3, NP), jnp.float32),
            jax.ShapeDtypeStruct((1, NP), jnp.float32),
        ],
    )(xT, degp, wcatT)


SPLIT_F = 16
SPLIT_E = NW // SPLIT_F
FPW = F3 // SPLIT_F
EPW_C = E // SPLIT_E
CH_C = 800
NCH_C = EPW_C // CH_C
NBUF = 2


def _msg_body(row_hbm, col_hbm, w_hbm, xwnT_hbm, accp_hbm,
              table_v, acc_v, row_v, col_v, w_v, sems):
    wid = lax.axis_index("s") * NC + lax.axis_index("c")
    fgroup = wid % SPLIT_F
    eslice = wid // SPLIT_F
    fbase = fgroup * FPW
    ebase = eslice * EPW_C

    def issue(ci, b):
        off = ebase + jnp.minimum(ci, NCH_C - 1) * CH_C
        pltpu.async_copy(row_hbm.at[pl.ds(off, CH_C)], row_v.at[b], sems.at[b])
        pltpu.async_copy(col_hbm.at[pl.ds(off, CH_C)], col_v.at[b], sems.at[b])
        pltpu.async_copy(w_hbm.at[pl.ds(off, CH_C)], w_v.at[b], sems.at[b])

    def drain(b):
        pltpu.make_async_copy(row_hbm.at[pl.ds(0, CH_C)], row_v.at[b], sems.at[b]).wait()
        pltpu.make_async_copy(col_hbm.at[pl.ds(0, CH_C)], col_v.at[b], sems.at[b]).wait()
        pltpu.make_async_copy(w_hbm.at[pl.ds(0, CH_C)], w_v.at[b], sems.at[b]).wait()

    for b in range(NBUF):
        issue(jnp.int32(b), b)

    pltpu.sync_copy(xwnT_hbm.at[pl.ds(fbase, FPW)], table_v)

    zero16 = jnp.zeros((16,), jnp.float32)
    for f in range(FPW):
        @plsc.parallel_loop(0, NP // 16)
        def _zero(j, f=f):
            acc_v[f, pl.ds(j * 16, 16)] = zero16

    @pl.loop(0, NCH_C, step=NBUF)
    def _chunk(g):
        for b in range(NBUF):
            ci = g + b
            drain(b)

            @plsc.parallel_loop(0, CH_C // 16, unroll=2)
            def _group(gg, b=b):
                o = gg * 16
                r16 = row_v[b, pl.ds(o, 16)]
                c16 = col_v[b, pl.ds(o, 16)]
                w16 = w_v[b, pl.ds(o, 16)]
                for f in range(FPW):
                    t = plsc.load_gather(table_v.at[f], [r16])
                    plsc.addupdate_scatter(acc_v.at[f], [c16], t * w16)

            issue(ci + NBUF, b)

    for b in range(NBUF):
        drain(b)
    pltpu.sync_copy(acc_v, accp_hbm.at[eslice, pl.ds(fbase, FPW)])


def _msg_partials(row, col, w, xwnT):
    mesh = plsc.VectorSubcoreMesh(core_axis_name="c", subcore_axis_name="s")
    return pl.kernel(
        _msg_body,
        compiler_params=pltpu.CompilerParams(needs_layout_passes=False, use_tc_tiling_on_sc=False),
        out_type=jax.ShapeDtypeStruct((SPLIT_E, F3, NP), jnp.float32),
        mesh=mesh,
        scratch_types=[
            pltpu.VMEM((FPW, NP), jnp.float32),
            pltpu.VMEM((FPW, NP), jnp.float32),
            pltpu.VMEM((NBUF, CH_C), jnp.int32),
            pltpu.VMEM((NBUF, CH_C), jnp.int32),
            pltpu.VMEM((NBUF, CH_C), jnp.float32),
            pltpu.SemaphoreType.DMA((NBUF,)),
        ],
    )(row, col, w, xwnT)


BN_D = 512


def _gates_body(accp_ref, xwnT_ref, dis_ref, hT_ref, bcat_ref,
                wz1_ref, wz2_ref, bz_ref, wr1_ref, wr2_ref, br_ref,
                wh1_ref, wh2_ref, bh_ref, wo_ref, bo_ref,
                hnT_ref, yT_ref):
    acc = accp_ref[0] + accp_ref[1]
    dis = dis_ref[...]
    S = dis * (acc + xwnT_ref[...]) + bcat_ref[...]
    cz = S[0:F_OUT]
    cr = S[F_OUT:2 * F_OUT]
    ch = S[2 * F_OUT:F3]
    H = hT_ref[...]

    def mm(a, b):
        return jnp.dot(a, b, preferred_element_type=jnp.float32)

    Z = jax.nn.sigmoid(mm(wz1_ref[...], cz) + mm(wz2_ref[...], H) + bz_ref[...])
    R = jax.nn.sigmoid(mm(wr1_ref[...], cr) + mm(wr2_ref[...], H) + br_ref[...])
    Ht = jnp.tanh(mm(wh1_ref[...], ch) + mm(wh2_ref[...], H * R) + bh_ref[...])
    Hn = Z * H + (1.0 - Z) * Ht
    hnT_ref[...] = Hn
    yT_ref[...] = mm(wo_ref[...], jnp.maximum(Hn, 0.0)) + bo_ref[...]


def _gates(accp, xwnT, dis, hT, bcat, wz1, wz2, bz, wr1, wr2, br,
           wh1, wh2, bh, wo, bo):
    grid = (NP // BN_D,)
    full = lambda shape: pl.BlockSpec(shape, lambda i: tuple(0 for _ in shape))
    return pl.pallas_call(
        _gates_body,
        grid=grid,
        in_specs=[
            pl.BlockSpec((SPLIT_E, F3, BN_D), lambda i: (0, 0, i)),
            pl.BlockSpec((F3, BN_D), lambda i: (0, i)),
            pl.BlockSpec((1, BN_D), lambda i: (0, i)),
            pl.BlockSpec((F_OUT, BN_D), lambda i: (0, i)),
            full((F3, 1)),
            full((F_OUT, F_OUT)), full((F_OUT, F_OUT)), full((F_OUT, 1)),
            full((F_OUT, F_OUT)), full((F_OUT, F_OUT)), full((F_OUT, 1)),
            full((F_OUT, F_OUT)), full((F_OUT, F_OUT)), full((F_OUT, 1)),
            full((1, F_OUT)), full((1, 1)),
        ],
        out_specs=[
            pl.BlockSpec((F_OUT, BN_D), lambda i: (0, i)),
            pl.BlockSpec((1, BN_D), lambda i: (0, i)),
        ],
        out_shape=[
            jax.ShapeDtypeStruct((F_OUT, NP), jnp.float32),
            jax.ShapeDtypeStruct((1, NP), jnp.float32),
        ],
    )(accp, xwnT, dis, hT, bcat, wz1, wz2, bz, wr1, wr2, br,
      wh1, wh2, bh, wo, bo)


def kernel(x, edge_index, edge_weight, prev_hidden_state,
           W_conv_z, b_conv_z, W_conv_r, b_conv_r, W_conv_h, b_conv_h,
           W_lin_z, b_lin_z, W_lin_r, b_lin_r, W_lin_h, b_lin_h,
           W_out, b_out):
    row = edge_index[0]
    col = edge_index[1]

    xT = jnp.zeros((F_IN, NP), jnp.float32).at[:, :N].set(x.T)
    hT = jnp.zeros((F_OUT, NP), jnp.float32).at[:, :N].set(prev_hidden_state.T)
    wcatT = jnp.concatenate([W_conv_z, W_conv_r, W_conv_h], axis=1).T
    bcat = jnp.concatenate([b_conv_z, b_conv_r, b_conv_h]).reshape(F3, 1)

    degp = _deg_partials(col, edge_weight)
    xwnT, dis = _xwn(xT, degp, wcatT)
    accp = _msg_partials(row, col, edge_weight, xwnT)

    hnT, yT = _gates(
        accp, xwnT, dis, hT, bcat,
        W_lin_z[:F_OUT].T, W_lin_z[F_OUT:].T, b_lin_z.reshape(F_OUT, 1),
        W_lin_r[:F_OUT].T, W_lin_r[F_OUT:].T, b_lin_r.reshape(F_OUT, 1),
        W_lin_h[:F_OUT].T, W_lin_h[F_OUT:].T, b_lin_h.reshape(F_OUT, 1),
        W_out.T, b_out.reshape(1, 1),
    )
    H_new = hnT[:, :N].T
    y = yT[:, :N].T
    return y, H_new
